# Initial kernel scaffold; baseline (speedup 1.0000x reference)
import jax
import jax.numpy as jnp
from jax.experimental import pallas as pl


def kernel(input_ids, token_type_ids, W, gamma, beta):
    raise NotImplementedError("write your pallas kernel here")


# speedup vs baseline: 2.6909x; 2.6909x over previous
import functools
import math

import jax
import jax.numpy as jnp
from jax import lax
from jax.experimental import pallas as pl
from jax.experimental.pallas import tpu as pltpu
from jax.experimental.pallas import tpu_sc as plsc

_NC = 2
_NS = 16
_NW = _NC * _NS

_VOCAB = 100000
_D = 128
_L = 200
_B = 1024
_TOK = _B * _L
_CH = 128
_CPW = _TOK // _NW
_NCHUNK = _CPW // _CH


def _allreduce_sum(v):
    lanes = lax.iota(jnp.int32, 16)
    dnums = lax.GatherDimensionNumbers(
        offset_dims=(), collapsed_slice_dims=(0,), start_index_map=(0,))
    for k in (8, 4, 2, 1):
        idx = lax.bitwise_xor(lanes, jnp.int32(k))
        v = v + lax.gather(v, idx[:, None], dnums, (1,),
                           mode=lax.GatherScatterMode.PROMISE_IN_BOUNDS)
    return v


def _rsqrt_nr(x):
    i = lax.bitcast_convert_type(x, jnp.int32)
    i = jnp.int32(0x5F3759DF) - lax.shift_right_logical(i, 1)
    r = lax.bitcast_convert_type(i, jnp.float32)
    for _ in range(3):
        r = r * (1.5 - 0.5 * x * r * r)
    return r


def _emb_ln_body(ids_hbm, pos_hbm, w_hbm, gam_hbm, bet_hbm, out_hbm,
                 ids_v, pos_v, gam_v, bet_v, rows_v, sem):
    cid = lax.axis_index("c")
    sid = lax.axis_index("s")
    wid = sid * _NC + cid
    gbase = wid * _CPW

    pltpu.sync_copy(ids_hbm.at[wid], ids_v)
    pltpu.sync_copy(pos_hbm, pos_v)
    pltpu.sync_copy(gam_hbm, gam_v)
    pltpu.sync_copy(bet_hbm, bet_v)

    def chunk_body(c, carry):
        pltpu.async_copy(w_hbm.at[ids_v.at[c]], rows_v, sem).wait()

        def tok_body(t, tcarry):
            pidx = lax.rem(c * _CH + t, _L)
            ys = []
            for j in range(8):
                r = rows_v[t, pl.ds(j * 16, 16)]
                p = pos_v[pidx, pl.ds(j * 16, 16)]
                ys.append(r + p)
            s = ys[0]
            for j in range(1, 8):
                s = s + ys[j]
            q = ys[0] * ys[0]
            for j in range(1, 8):
                q = q + ys[j] * ys[j]
            mean = _allreduce_sum(s) * (1.0 / _D)
            msq = _allreduce_sum(q) * (1.0 / _D)
            var = msq - mean * mean
            rstd = _rsqrt_nr(var + 1e-6)
            for j in range(8):
                g = gam_v[pl.ds(j * 16, 16)]
                b = bet_v[pl.ds(j * 16, 16)]
                rows_v[t, pl.ds(j * 16, 16)] = (ys[j] - mean) * rstd * g + b
            return tcarry

        lax.fori_loop(0, _CH, tok_body, 0)
        pltpu.sync_copy(rows_v, out_hbm.at[pl.ds(gbase + c * _CH, _CH)])
        return carry

    lax.fori_loop(0, _NCHUNK, chunk_body, 0)


_emb_ln = functools.partial(
    pl.kernel,
    out_type=jax.ShapeDtypeStruct((_TOK, _D), jnp.float32),
    mesh=plsc.VectorSubcoreMesh(core_axis_name="c", subcore_axis_name="s",
                                num_cores=_NC, num_subcores=_NS),
    scratch_types=[
        pltpu.VMEM((_NCHUNK, _CH), jnp.int32),
        pltpu.VMEM((_L, _D), jnp.float32),
        pltpu.VMEM((_D,), jnp.float32),
        pltpu.VMEM((_D,), jnp.float32),
        pltpu.VMEM((_CH, _D), jnp.float32),
        pltpu.SemaphoreType.DMA,
    ],
)(_emb_ln_body)


def _pos_table(length, d):
    pos = jnp.arange(length, dtype=jnp.float32)[:, None]
    i = jnp.arange(d, dtype=jnp.float32)[None, :]
    angles = pos * (1.0 / jnp.power(10000.0, 2.0 * jnp.floor(i / 2.0) / jnp.float32(d)))
    pe = jnp.zeros((length, d), dtype=jnp.float32)
    pe = pe.at[:, 0::2].set(jnp.sin(angles[:, 0::2]))
    pe = pe.at[:, 1::2].set(jnp.cos(angles[:, 1::2]))
    return pe


def kernel(input_ids, token_type_ids, W, gamma, beta):
    del token_type_ids
    ids = input_ids.reshape(_NW, _NCHUNK, _CH)
    pe = _pos_table(_L, _D) * jnp.float32(1.0 / math.sqrt(_D))
    out = _emb_ln(ids, pe, W, gamma, beta)
    return out.reshape(_B, _L, _D)
